# baseline (speedup 1.0000x reference)
import functools

import jax
import jax.numpy as jnp
from jax.experimental import pallas as pl
from jax.experimental.pallas import tpu as pltpu

NUM_CLUSTERS = 512
NUM_ITERS = 20
TILE = 256


def _assign_kernel(n_real, pts_ref, cent_ref, out_ref):
    npad = pts_ref.shape[2]
    ntiles = npad // TILE

    def _bf(v):
        return v.astype(jnp.bfloat16).astype(jnp.float32)

    cx = cent_ref[0, :, 0:1]
    cy = cent_ref[0, :, 1:2]
    cxb = _bf(cx)
    cyb = _bf(cy)
    c2 = cx * cx + cy * cy

    def tile_body(t, _):
        px = pts_ref[0, 0:1, pl.ds(t * TILE, TILE)]
        py = pts_ref[0, 1:2, pl.ds(t * TILE, TILE)]
        x2 = px * px + py * py
        dot = _bf(px) * cxb + _bf(py) * cyb
        d = jnp.sqrt(jnp.maximum((x2 + c2) - 2.0 * dot, 0.0))
        ids = jnp.argmin(d, axis=0).reshape(1, TILE)
        out_ref[0, 0:1, pl.ds(t * TILE, TILE)] = ids
        return 0

    jax.lax.fori_loop(0, ntiles, tile_body, 0)


def _assign(pts, cent, n, npad):
    b = pts.shape[0]
    out = pl.pallas_call(
        functools.partial(_assign_kernel, n),
        grid=(b,),
        in_specs=[
            pl.BlockSpec((1, 2, npad), lambda i: (i, 0, 0)),
            pl.BlockSpec((1, NUM_CLUSTERS, 2), lambda i: (i, 0, 0)),
        ],
        out_specs=pl.BlockSpec((1, 1, npad), lambda i: (i, 0, 0)),
        out_shape=jax.ShapeDtypeStruct((b, 1, npad), jnp.int32),
    )(pts, cent)
    return out[:, 0, :n]


def kernel(coords):
    b, n, d = coords.shape
    npad = ((n + TILE - 1) // TILE) * TILE

    cents = []
    for i in range(b):
        key = jax.random.fold_in(jax.random.key(1), i)
        idx = jax.random.permutation(key, n)[:NUM_CLUSTERS]
        cents.append(coords[i, idx])
    cent = jnp.stack(cents, axis=0)

    pts = jnp.transpose(coords, (0, 2, 1))
    pts = jnp.pad(pts, ((0, 0), (0, 0), (0, npad - n)))

    ids = None
    for _ in range(NUM_ITERS):
        ids = _assign(pts, cent, n, npad)
        new_cent = []
        for i in range(b):
            pts_i = coords[i]
            ids_i = ids[i]
            sums = jax.ops.segment_sum(pts_i, ids_i,
                                       num_segments=NUM_CLUSTERS)
            counts = jax.ops.segment_sum(
                jnp.ones((n,), dtype=pts_i.dtype), ids_i,
                num_segments=NUM_CLUSTERS)
            new_cent.append(jnp.where(
                counts[:, None] > 0,
                sums / jnp.maximum(counts[:, None], 1.0), cent[i]))
        cent = jnp.stack(new_cent, axis=0)
    return ids


# speedup vs baseline: 1.0926x; 1.0926x over previous
import functools

import jax
import jax.numpy as jnp
from jax.experimental import pallas as pl
from jax.experimental.pallas import tpu as pltpu

NUM_CLUSTERS = 512
NUM_ITERS = 20
TILE = 256


def _assign_kernel(n_real, pts_ref, cent_ref, out_ref, cnt_ref, acc_ref):
    npad = pts_ref.shape[2]
    ntiles = npad // TILE

    def _bf(v):
        return v.astype(jnp.bfloat16).astype(jnp.float32)

    cx = cent_ref[0, :, 0:1]
    cy = cent_ref[0, :, 1:2]
    cxb = _bf(cx)
    cyb = _bf(cy)
    c2 = cx * cx + cy * cy

    sub_iota = jax.lax.broadcasted_iota(jnp.int32, (NUM_CLUSTERS, 1), 0)
    lane_iota = jax.lax.broadcasted_iota(jnp.int32, (1, TILE), 1)

    acc_ref[...] = jnp.zeros_like(acc_ref)

    def tile_ids(t):
        px = pts_ref[0, 0:1, pl.ds(t * TILE, TILE)]
        py = pts_ref[0, 1:2, pl.ds(t * TILE, TILE)]
        x2 = px * px + py * py
        dot = _bf(px) * cxb + _bf(py) * cyb
        d = jnp.sqrt(jnp.maximum((x2 + c2) - 2.0 * dot, 0.0))
        minv = jnp.min(d, axis=0, keepdims=True)
        return jnp.min(jnp.where(d == minv, sub_iota, NUM_CLUSTERS),
                       axis=0).reshape(1, TILE)

    def count_tile(t, masked):
        ids = tile_ids(t)
        out_ref[0, 0:1, pl.ds(t * TILE, TILE)] = ids
        oh = ids == sub_iota
        if masked:
            valid = (lane_iota + t * TILE) < n_real
            oh = jnp.logical_and(oh, valid)
        acc_ref[:, 0:1] += jnp.sum(
            jnp.where(oh, 1.0, 0.0).astype(jnp.float32),
            axis=1, keepdims=True)

    nfull = min(n_real // TILE, ntiles)

    def tile_body(t, _):
        count_tile(t, masked=False)
        return 0

    jax.lax.fori_loop(0, nfull, tile_body, 0)
    for t_tail in range(nfull, ntiles):
        count_tile(t_tail, masked=True)

    cnt_ref[0] = acc_ref[...]


def _assign(pts, cent, n, npad):
    b = pts.shape[0]
    ids, cnt = pl.pallas_call(
        functools.partial(_assign_kernel, n),
        grid=(b,),
        in_specs=[
            pl.BlockSpec((1, 2, npad), lambda i: (i, 0, 0)),
            pl.BlockSpec((1, NUM_CLUSTERS, 2), lambda i: (i, 0, 0)),
        ],
        out_specs=(
            pl.BlockSpec((1, 1, npad), lambda i: (i, 0, 0)),
            pl.BlockSpec((1, NUM_CLUSTERS, 1), lambda i: (i, 0, 0)),
        ),
        out_shape=(
            jax.ShapeDtypeStruct((b, 1, npad), jnp.int32),
            jax.ShapeDtypeStruct((b, NUM_CLUSTERS, 1), jnp.float32),
        ),
        scratch_shapes=[pltpu.VMEM((NUM_CLUSTERS, 1), jnp.float32)],
    )(pts, cent)
    return ids[:, 0, :n], cnt[:, :, 0]


def kernel(coords):
    b, n, d = coords.shape
    npad = ((n + TILE - 1) // TILE) * TILE

    cents = []
    for i in range(b):
        key = jax.random.fold_in(jax.random.key(1), i)
        idx = jax.random.permutation(key, n)[:NUM_CLUSTERS]
        cents.append(coords[i, idx])
    cent = jnp.stack(cents, axis=0)

    pts = jnp.transpose(coords, (0, 2, 1))
    pts = jnp.pad(pts, ((0, 0), (0, 0), (0, npad - n)))

    ids = None
    for _ in range(NUM_ITERS):
        ids, counts = _assign(pts, cent, n, npad)
        new_cent = []
        for i in range(b):
            sums = jax.ops.segment_sum(coords[i], ids[i],
                                       num_segments=NUM_CLUSTERS)
            new_cent.append(jnp.where(
                counts[i][:, None] > 0,
                sums / jnp.maximum(counts[i][:, None], 1.0), cent[i]))
        cent = jnp.stack(new_cent, axis=0)
    return ids


# speedup vs baseline: 1.6821x; 1.5396x over previous
import functools

import jax
import jax.numpy as jnp
from jax.experimental import pallas as pl
from jax.experimental.pallas import tpu as pltpu

NUM_CLUSTERS = 512
NUM_ITERS = 20
TILE = 256


def _assign_kernel(n_real, pts_ref, cent_ref, out_ref, cnt_ref, acc_ref):
    npad = pts_ref.shape[2]
    ntiles = npad // TILE

    def _bf(v):
        return v.astype(jnp.bfloat16).astype(jnp.float32)

    cx = cent_ref[0, :, 0:1]
    cy = cent_ref[0, :, 1:2]
    cxb = _bf(cx)
    cyb = _bf(cy)
    c2 = cx * cx + cy * cy

    sub_iota = jax.lax.broadcasted_iota(jnp.int32, (NUM_CLUSTERS, 1), 0)
    lane_iota = jax.lax.broadcasted_iota(jnp.int32, (1, TILE), 1)

    acc_ref[...] = jnp.zeros_like(acc_ref)

    def tile_ids(t):
        px = pts_ref[0, 0:1, pl.ds(t * TILE, TILE)]
        py = pts_ref[0, 1:2, pl.ds(t * TILE, TILE)]
        x2 = px * px + py * py
        dot = _bf(px) * cxb + _bf(py) * cyb
        d = jnp.sqrt(jnp.maximum((x2 + c2) - 2.0 * dot, 0.0))
        minv = jnp.min(d, axis=0, keepdims=True)
        return jnp.min(jnp.where(d == minv, sub_iota, NUM_CLUSTERS),
                       axis=0).reshape(1, TILE)

    def count_tile(t, masked):
        ids = tile_ids(t)
        out_ref[0, 0:1, pl.ds(t * TILE, TILE)] = ids
        oh = ids == sub_iota
        if masked:
            valid = (lane_iota + t * TILE) < n_real
            oh = jnp.logical_and(oh, valid)
        acc_ref[:, 0:1] += jnp.sum(
            jnp.where(oh, 1.0, 0.0).astype(jnp.float32),
            axis=1, keepdims=True)

    nfull = min(n_real // TILE, ntiles)

    def tile_body(t, _):
        count_tile(t, masked=False)
        return 0

    jax.lax.fori_loop(0, nfull, tile_body, 0)
    for t_tail in range(nfull, ntiles):
        count_tile(t_tail, masked=True)

    cnt_ref[0] = acc_ref[...]


def _assign_one(pts_i, cent_i, n, npad):
    ids, cnt = pl.pallas_call(
        functools.partial(_assign_kernel, n),
        grid=(1,),
        in_specs=[
            pl.BlockSpec((1, 2, npad), lambda i: (0, 0, 0)),
            pl.BlockSpec((1, NUM_CLUSTERS, 2), lambda i: (0, 0, 0)),
        ],
        out_specs=(
            pl.BlockSpec((1, 1, npad), lambda i: (0, 0, 0)),
            pl.BlockSpec((1, NUM_CLUSTERS, 1), lambda i: (0, 0, 0)),
        ),
        out_shape=(
            jax.ShapeDtypeStruct((1, 1, npad), jnp.int32),
            jax.ShapeDtypeStruct((1, NUM_CLUSTERS, 1), jnp.float32),
        ),
        scratch_shapes=[pltpu.VMEM((NUM_CLUSTERS, 1), jnp.float32)],
    )(pts_i[None], cent_i[None])
    return ids[0, 0, :n], cnt[0, :, 0]


def kernel(coords):
    b, n, d = coords.shape
    npad = ((n + TILE - 1) // TILE) * TILE

    cents = []
    for i in range(b):
        key = jax.random.fold_in(jax.random.key(1), i)
        idx = jax.random.permutation(key, n)[:NUM_CLUSTERS]
        cents.append(coords[i, idx])
    cent = jnp.stack(cents, axis=0)

    pts = jnp.transpose(coords, (0, 2, 1))
    pts = jnp.pad(pts, ((0, 0), (0, 0), (0, npad - n)))

    cent_l = [cent[i] for i in range(b)]
    ids_l = [None] * b
    for _ in range(NUM_ITERS):
        for i in range(b):
            ids_l[i], cnt_i = _assign_one(pts[i], cent_l[i], n, npad)
            sums = jax.ops.segment_sum(coords[i], ids_l[i],
                                       num_segments=NUM_CLUSTERS)
            cent_l[i] = jnp.where(
                cnt_i[:, None] > 0,
                sums / jnp.maximum(cnt_i[:, None], 1.0), cent_l[i])
    return jnp.stack(ids_l, axis=0)
